# baseline (speedup 1.0000x reference)
import jax
import jax.numpy as jnp
from jax import lax
from jax.experimental import pallas as pl
from jax.experimental.pallas import tpu as pltpu
from jax.experimental.pallas import tpu_sc as plsc

DIM = 64
PAD = 128
SCALE = 8.0

_info = plsc.get_sparse_core_info()
NC, NS = _info.num_cores, _info.num_subcores
NW = NC * NS

R = 2
LANES = 16


def _emb_body(table_hbm, idx_hbm, out_hbm, idx_v, rows_v, padded_v, sem):
    n_x_rows, L = idx_hbm.shape
    rows_per_w = n_x_rows // NW
    n_chunks = rows_per_w // R
    segs = []
    off = 0
    while off < L:
        w = min(128, L - off)
        segs.append((off, w))
        off += w
    wid = lax.axis_index("s") * NC + lax.axis_index("c")
    row_base = wid * rows_per_w

    pltpu.sync_copy(idx_hbm.at[pl.ds(row_base, rows_per_w)], idx_v)

    def chunk_body(ci, carry):
        row_off = ci * R
        copies = [
            pltpu.async_copy(
                table_hbm.at[idx_v.at[row_off + r, pl.ds(s_off, s_w)]],
                rows_v.at[pl.ds(r * L + s_off, s_w)],
                sem,
            )
            for r in range(R)
            for (s_off, s_w) in segs
        ]
        for c in copies:
            c.wait()

        def copy_body(r, acc):
            for c0 in range(0, DIM, LANES):
                padded_v[r, pl.ds(c0, LANES)] = rows_v[r, pl.ds(c0, LANES)]
            return acc

        lax.fori_loop(0, R * L, copy_body, 0)
        pltpu.sync_copy(
            padded_v, out_hbm.at[pl.ds((row_base + row_off) * L, R * L)]
        )
        return carry

    lax.fori_loop(0, n_chunks, chunk_body, 0)


def kernel(x, table):
    B, L = x.shape
    rows_per_w = B // NW
    mesh = plsc.VectorSubcoreMesh(core_axis_name="c", subcore_axis_name="s")
    run = pl.kernel(
        _emb_body,
        mesh=mesh,
        compiler_params=pltpu.CompilerParams(
            use_tc_tiling_on_sc=False, needs_layout_passes=False
        ),
        out_type=jax.ShapeDtypeStruct((B * L, PAD), jnp.float32),
        scratch_types=[
            pltpu.VMEM((rows_per_w, L), jnp.int32),
            pltpu.VMEM((R * L, DIM), jnp.float32),
            pltpu.VMEM((R * L, PAD), jnp.float32),
            pltpu.SemaphoreType.DMA,
        ],
    )
    out2 = run(table * SCALE, x)
    return out2[:, :DIM].reshape(B, L, DIM)


# speedup vs baseline: 1.6293x; 1.6293x over previous
import jax
import jax.numpy as jnp
from jax import lax
from jax.experimental import pallas as pl
from jax.experimental.pallas import tpu as pltpu
from jax.experimental.pallas import tpu_sc as plsc

DIM = 64
PAD = 128
SCALE = 8.0
LANES = 16

_info = plsc.get_sparse_core_info()
NC, NS = _info.num_cores, _info.num_subcores
NW = NC * NS

R = 4


def _emb_body(table_hbm, idx_hbm, out_hbm, idx_v, rows_v, sem):
    n_x_rows, L = idx_hbm.shape
    rows_per_w = n_x_rows // NW
    n_chunks = rows_per_w // R
    segs = []
    off = 0
    while off < L:
        w = min(128, L - off)
        segs.append((off, w))
        off += w
    wid = lax.axis_index("s") * NC + lax.axis_index("c")
    row_base = wid * rows_per_w

    pltpu.sync_copy(idx_hbm.at[pl.ds(row_base, rows_per_w)], idx_v)

    def chunk_body(ci, carry):
        row_off = ci * R
        copies = [
            pltpu.async_copy(
                table_hbm.at[idx_v.at[row_off + r, pl.ds(s_off, s_w)]],
                rows_v.at[pl.ds(r * L + s_off, s_w)],
                sem,
            )
            for r in range(R)
            for (s_off, s_w) in segs
        ]
        for c in copies:
            c.wait()

        def scale_body(r, acc):
            for c0 in range(0, DIM, LANES):
                rows_v[r, pl.ds(c0, LANES)] = rows_v[r, pl.ds(c0, LANES)] * SCALE
            return acc

        lax.fori_loop(0, R * L, scale_body, 0)
        pltpu.sync_copy(
            rows_v,
            out_hbm.at[pl.ds((row_base + row_off) * L, R * L), pl.ds(0, DIM)],
        )
        return carry

    lax.fori_loop(0, n_chunks, chunk_body, 0)


def kernel(x, table):
    B, L = x.shape
    rows_per_w = B // NW
    mesh = plsc.VectorSubcoreMesh(core_axis_name="c", subcore_axis_name="s")
    run = pl.kernel(
        _emb_body,
        mesh=mesh,
        compiler_params=pltpu.CompilerParams(
            use_tc_tiling_on_sc=False, needs_layout_passes=False
        ),
        out_type=jax.ShapeDtypeStruct((B * L, PAD), jnp.float32),
        scratch_types=[
            pltpu.VMEM((rows_per_w, L), jnp.int32),
            pltpu.VMEM((R * L, DIM), jnp.float32),
            pltpu.SemaphoreType.DMA,
        ],
    )
    out2 = run(table, x)
    return out2[:, :DIM].reshape(B, L, DIM)


# speedup vs baseline: 1.7762x; 1.0902x over previous
import jax
import jax.numpy as jnp
from jax import lax
from jax.experimental import pallas as pl
from jax.experimental.pallas import tpu as pltpu
from jax.experimental.pallas import tpu_sc as plsc

DIM = 64
PAD = 128
SCALE = 8.0
LANES = 16

_info = plsc.get_sparse_core_info()
NC, NS = _info.num_cores, _info.num_subcores
NW = NC * NS

R = 4


def _emb_body(table_hbm, idx_hbm, out_hbm, idx_v, rows_v, sem):
    n_x_rows, L = idx_hbm.shape
    rows_per_w = n_x_rows // NW
    n_chunks = rows_per_w // R
    segs = []
    off = 0
    while off < L:
        w = min(128, L - off)
        segs.append((off, w))
        off += w
    wid = lax.axis_index("s") * NC + lax.axis_index("c")
    row_base = wid * rows_per_w

    pltpu.sync_copy(idx_hbm.at[pl.ds(row_base, rows_per_w)], idx_v)

    def fire(ci, buf, sem):
        row_off = ci * R
        return [
            pltpu.async_copy(
                table_hbm.at[idx_v.at[row_off + r, pl.ds(s_off, s_w)]],
                buf.at[pl.ds(r * L + s_off, s_w)],
                sem,
            )
            for r in range(R)
            for (s_off, s_w) in segs
        ]

    def drain(buf, sem):
        for r in range(R):
            for (s_off, s_w) in segs:
                pltpu.make_async_copy(
                    table_hbm.at[idx_v.at[r, pl.ds(s_off, s_w)]],
                    buf.at[pl.ds(r * L + s_off, s_w)],
                    sem,
                ).wait()

    def scale_and_store(ci, buf):
        def scale_body(r, acc):
            for c0 in range(0, DIM, LANES):
                buf[r, pl.ds(c0, LANES)] = buf[r, pl.ds(c0, LANES)] * SCALE
            return acc

        lax.fori_loop(0, R * L, scale_body, 0)
        pltpu.sync_copy(
            buf,
            out_hbm.at[pl.ds((row_base + ci * R) * L, R * L), pl.ds(0, DIM)],
        )

    bufs = (rows_v.at[0], rows_v.at[1])
    sems = (sem.at[0], sem.at[1])
    fire(0, bufs[0], sems[0])
    fire(1, bufs[1], sems[1])

    def chunk_body(c2, carry):
        c = c2 * 2
        drain(bufs[0], sems[0])
        scale_and_store(c, bufs[0])
        fire(c + 2, bufs[0], sems[0])
        drain(bufs[1], sems[1])
        scale_and_store(c + 1, bufs[1])
        fire(c + 3, bufs[1], sems[1])
        return carry

    lax.fori_loop(0, n_chunks // 2 - 1, chunk_body, 0)
    drain(bufs[0], sems[0])
    scale_and_store(n_chunks - 2, bufs[0])
    drain(bufs[1], sems[1])
    scale_and_store(n_chunks - 1, bufs[1])


def kernel(x, table):
    B, L = x.shape
    rows_per_w = B // NW
    mesh = plsc.VectorSubcoreMesh(core_axis_name="c", subcore_axis_name="s")
    run = pl.kernel(
        _emb_body,
        mesh=mesh,
        compiler_params=pltpu.CompilerParams(
            use_tc_tiling_on_sc=False, needs_layout_passes=False
        ),
        out_type=jax.ShapeDtypeStruct((B * L, PAD), jnp.float32),
        scratch_types=[
            pltpu.VMEM((rows_per_w, L), jnp.int32),
            pltpu.VMEM((2, R * L, DIM), jnp.float32),
            pltpu.SemaphoreType.DMA((2,)),
        ],
    )
    out2 = run(table, x)
    return out2[:, :DIM].reshape(B, L, DIM)


# speedup vs baseline: 1.8414x; 1.0367x over previous
import jax
import jax.numpy as jnp
from jax import lax
from jax.experimental import pallas as pl
from jax.experimental.pallas import tpu as pltpu
from jax.experimental.pallas import tpu_sc as plsc

DIM = 64
PAD = 128
SCALE = 8.0
LANES = 16

_info = plsc.get_sparse_core_info()
NC, NS = _info.num_cores, _info.num_subcores
NW = NC * NS

R = 4


def _emb_body(table_hbm, idx_hbm, out_hbm, idx_v, rows_v, sem):
    n_x_rows, L = idx_hbm.shape
    rows_per_w = n_x_rows // NW
    n_chunks = rows_per_w // R
    segs = []
    off = 0
    while off < L:
        w = min(128, L - off)
        segs.append((off, w))
        off += w
    wid = lax.axis_index("s") * NC + lax.axis_index("c")
    row_base = wid * rows_per_w

    pltpu.sync_copy(idx_hbm.at[pl.ds(row_base, rows_per_w)], idx_v)

    def fire(ci, buf, sem):
        row_off = ci * R
        return [
            pltpu.async_copy(
                table_hbm.at[idx_v.at[row_off + r, pl.ds(s_off, s_w)]],
                buf.at[pl.ds(r * L + s_off, s_w)],
                sem,
            )
            for r in range(R)
            for (s_off, s_w) in segs
        ]

    def drain(buf, sem):
        for r in range(R):
            for (s_off, s_w) in segs:
                pltpu.make_async_copy(
                    table_hbm.at[idx_v.at[r, pl.ds(s_off, s_w)]],
                    buf.at[pl.ds(r * L + s_off, s_w)],
                    sem,
                ).wait()

    def scale_and_store(ci, buf):
        def scale_body(r4, acc):
            r = r4 * 4
            for dr in range(4):
                for c0 in range(0, DIM, LANES):
                    buf[r + dr, pl.ds(c0, LANES)] = (
                        buf[r + dr, pl.ds(c0, LANES)] * SCALE
                    )
            return acc

        lax.fori_loop(0, (R * L) // 4, scale_body, 0)
        pltpu.sync_copy(
            buf,
            out_hbm.at[pl.ds((row_base + ci * R) * L, R * L), pl.ds(0, DIM)],
        )

    bufs = (rows_v.at[0], rows_v.at[1])
    sems = (sem.at[0], sem.at[1])
    fire(0, bufs[0], sems[0])
    fire(1, bufs[1], sems[1])

    def chunk_body(c2, carry):
        c = c2 * 2
        drain(bufs[0], sems[0])
        scale_and_store(c, bufs[0])
        fire(c + 2, bufs[0], sems[0])
        drain(bufs[1], sems[1])
        scale_and_store(c + 1, bufs[1])
        fire(c + 3, bufs[1], sems[1])
        return carry

    lax.fori_loop(0, n_chunks // 2 - 1, chunk_body, 0)
    drain(bufs[0], sems[0])
    scale_and_store(n_chunks - 2, bufs[0])
    drain(bufs[1], sems[1])
    scale_and_store(n_chunks - 1, bufs[1])


def kernel(x, table):
    B, L = x.shape
    rows_per_w = B // NW
    mesh = plsc.VectorSubcoreMesh(core_axis_name="c", subcore_axis_name="s")
    run = pl.kernel(
        _emb_body,
        mesh=mesh,
        compiler_params=pltpu.CompilerParams(
            use_tc_tiling_on_sc=False, needs_layout_passes=False
        ),
        out_type=jax.ShapeDtypeStruct((B * L, PAD), jnp.float32),
        scratch_types=[
            pltpu.VMEM((rows_per_w, L), jnp.int32),
            pltpu.VMEM((2, R * L, DIM), jnp.float32),
            pltpu.SemaphoreType.DMA((2,)),
        ],
    )
    out2 = run(table, x)
    return out2[:, :DIM].reshape(B, L, DIM)
